# baseline (speedup 1.0000x reference)
import jax
import jax.numpy as jnp
from jax import lax
from jax.experimental import pallas as pl
from jax.experimental.pallas import tpu as pltpu
from jax.experimental.pallas import tpu_sc as plsc

_D = 64
_B = 4096
_H = 200
_NW = 32
_LT = _B // _NW
_NBUF = 2


def _body(table, x4, out, idxb, g2, outb, *sems):
    gsem = sems[:_NBUF]
    osem = sems[_NBUF:]
    w = lax.axis_index("s") * 2 + lax.axis_index("c")

    pltpu.sync_copy(x4.at[:, w], idxb)

    iota = lax.iota(jnp.int32, 16)

    def _issue_gather(ht, hs, b):
        pltpu.async_copy(table.at[idxb.at[ht, hs]], g2.at[b], gsem[b])

    def _transpose(b):
        for g in range(8):
            rvec = iota + g * 16

            @pl.loop(0, _D, unroll=8)
            def _dloop(d):
                vals = plsc.load_gather(g2.at[b], [rvec, iota * 0 + d])
                outb[b, lax.shift_right_logical(d, 3),
                     lax.bitwise_and(d, 7), pl.ds(g * 16, 16)] = vals

    for b in range(_NBUF):
        _issue_gather(0, b, b)

    @pl.loop(0, _H // 8)
    def _chunk(ht):
        h0 = ht * 8
        for hs in range(8):
            b = hs % _NBUF
            h = h0 + hs
            o = out.at[h, :, w]
            pltpu.make_async_copy(
                table.at[idxb.at[ht, hs]], g2.at[b], gsem[b]).wait()

            if hs >= _NBUF:
                pltpu.make_async_copy(outb.at[b], o, osem[b]).wait()
            else:
                @pl.when(ht > 0)
                def _():
                    pltpu.make_async_copy(outb.at[b], o, osem[b]).wait()

            _transpose(b)
            pltpu.async_copy(outb.at[b], o, osem[b])

            if hs < 6:
                _issue_gather(ht, hs + 2, b)
            else:
                @pl.when(ht < _H // 8 - 1)
                def _():
                    _issue_gather(ht + 1, hs - 6, b)

    for b in range(_NBUF):
        o = out.at[_H - _NBUF + b, :, w]
        pltpu.make_async_copy(outb.at[b], o, osem[b]).wait()


_mesh = plsc.VectorSubcoreMesh(core_axis_name="c", subcore_axis_name="s")

_gather = pl.kernel(
    _body,
    out_type=jax.ShapeDtypeStruct((_H, 8, _NW, 8, 128), jnp.float32),
    mesh=_mesh,
    scratch_types=[
        pltpu.VMEM((_H // 8, 8, _LT), jnp.int32),
        pltpu.VMEM((_NBUF, _LT, 128), jnp.float32),
        pltpu.VMEM((_NBUF, 8, 8, 128), jnp.float32),
    ] + [pltpu.SemaphoreType.DMA] * (2 * _NBUF),
    compiler_params=pltpu.CompilerParams(
        use_tc_tiling_on_sc=False, needs_layout_passes=False),
)


@jax.jit
def kernel(x, embedding):
    x4 = jnp.transpose(x).astype(jnp.int32).reshape(25, 8, 32, 128)
    x4 = jnp.transpose(x4, (0, 2, 1, 3))
    table = jnp.pad(embedding, ((0, 0), (0, 64)))
    o5 = _gather(table, x4)
    out = jnp.transpose(o5, (2, 4, 0, 1, 3))
    return out.reshape(_B, _H, _D)


# speedup vs baseline: 1.7431x; 1.7431x over previous
import jax
import jax.numpy as jnp
from jax import lax
from jax.experimental import pallas as pl
from jax.experimental.pallas import tpu as pltpu
from jax.experimental.pallas import tpu_sc as plsc

_D = 64
_B = 4096
_H = 200
_N = _B * _H
_NW = 32
_PER_W = _N // _NW
_CH = 128
_NCHUNK = _PER_W // _CH
_NBUF = 4
_NGROUP = _NCHUNK // _NBUF


def _body(table, idx, out, idxb, rows, *sems):
    gsem = sems[:_NBUF]
    psem = sems[_NBUF:]
    w = lax.axis_index("s") * 2 + lax.axis_index("c")
    base = w * _PER_W

    pltpu.sync_copy(idx.at[w], idxb)

    for b in range(_NBUF):
        pltpu.async_copy(table.at[idxb.at[b]], rows.at[b], gsem[b])

    @pl.loop(0, _NGROUP - 1)
    def _group(g):
        for b in range(_NBUF):
            j = g * _NBUF + b
            o = out.at[pl.ds(base + j * _CH, _CH)]
            pltpu.make_async_copy(
                table.at[idxb.at[j]], rows.at[b], gsem[b]).wait()
            pltpu.async_copy(rows.at[b], o, psem[b])
            pltpu.make_async_copy(rows.at[b], o, psem[b]).wait()
            pltpu.async_copy(table.at[idxb.at[j + _NBUF]], rows.at[b], gsem[b])

    for b in range(_NBUF):
        j = (_NGROUP - 1) * _NBUF + b
        o = out.at[pl.ds(base + j * _CH, _CH)]
        pltpu.make_async_copy(table.at[idxb.at[j]], rows.at[b], gsem[b]).wait()
        pltpu.async_copy(rows.at[b], o, psem[b])
    for b in range(_NBUF):
        j = (_NGROUP - 1) * _NBUF + b
        o = out.at[pl.ds(base + j * _CH, _CH)]
        pltpu.make_async_copy(rows.at[b], o, psem[b]).wait()


_mesh = plsc.VectorSubcoreMesh(core_axis_name="c", subcore_axis_name="s")

_gather = pl.kernel(
    _body,
    out_type=jax.ShapeDtypeStruct((_N, 128), jnp.float32),
    mesh=_mesh,
    scratch_types=[
        pltpu.VMEM((_NCHUNK, _CH), jnp.int32),
        pltpu.VMEM((_NBUF, _CH, 128), jnp.float32),
    ] + [pltpu.SemaphoreType.DMA] * (2 * _NBUF),
)


@jax.jit
def kernel(x, embedding):
    idx = x.astype(jnp.int32).reshape(_NW, _NCHUNK, _CH)
    table = jnp.pad(embedding, ((0, 0), (0, 64)))
    out = _gather(table, idx)
    return out.reshape(_B, _H, 128)[:, :, :_D]


# speedup vs baseline: 1.8494x; 1.0610x over previous
import jax
import jax.numpy as jnp
from jax import lax
from jax.experimental import pallas as pl
from jax.experimental.pallas import tpu as pltpu
from jax.experimental.pallas import tpu_sc as plsc

_D = 64
_B = 4096
_H = 200
_N = _B * _H
_NW = 32
_PER_W = _N // _NW
_CH = 128
_NCHUNK = _PER_W // _CH
_NBUF = 4
_NGROUP = _NCHUNK // _NBUF


def _body(table, idx, out, idxb, rows, *sems):
    gsem = sems[:_NBUF]
    psem = sems[_NBUF:]
    w = lax.axis_index("s") * 2 + lax.axis_index("c")
    base = w * _PER_W

    pltpu.sync_copy(idx.at[w], idxb)

    for b in range(_NBUF):
        pltpu.async_copy(table.at[idxb.at[b]], rows.at[b], gsem[b])

    @pl.loop(0, _NGROUP - 1)
    def _group(g):
        for b in range(_NBUF):
            j = g * _NBUF + b
            o = out.at[pl.ds(base + j * _CH, _CH)]
            pltpu.make_async_copy(
                table.at[idxb.at[j]], rows.at[b], gsem[b]).wait()
            pltpu.async_copy(rows.at[b], o, psem[b])
            pltpu.make_async_copy(rows.at[b], o, psem[b]).wait()
            pltpu.async_copy(table.at[idxb.at[j + _NBUF]], rows.at[b], gsem[b])

    for b in range(_NBUF):
        j = (_NGROUP - 1) * _NBUF + b
        o = out.at[pl.ds(base + j * _CH, _CH)]
        pltpu.make_async_copy(table.at[idxb.at[j]], rows.at[b], gsem[b]).wait()
        pltpu.async_copy(rows.at[b], o, psem[b])
    for b in range(_NBUF):
        j = (_NGROUP - 1) * _NBUF + b
        o = out.at[pl.ds(base + j * _CH, _CH)]
        pltpu.make_async_copy(rows.at[b], o, psem[b]).wait()


_TW = 2048


def _tp_body(tt_ref, out_ref):
    blk = tt_ref[...]
    t = jnp.transpose(blk, (1, 0))
    out_ref[...] = jnp.pad(t, ((0, 0), (0, 64)))


_tc_pad = pl.pallas_call(
    _tp_body,
    out_shape=jax.ShapeDtypeStruct((1000000, 128), jnp.float32),
    grid=(pl.cdiv(1000000, _TW),),
    in_specs=[pl.BlockSpec((64, _TW), lambda i: (0, i))],
    out_specs=pl.BlockSpec((_TW, 128), lambda i: (i, 0)),
)


_mesh = plsc.VectorSubcoreMesh(core_axis_name="c", subcore_axis_name="s")

_gather = pl.kernel(
    _body,
    out_type=jax.ShapeDtypeStruct((_N, 128), jnp.float32),
    mesh=_mesh,
    scratch_types=[
        pltpu.VMEM((_NCHUNK, _CH), jnp.int32),
        pltpu.VMEM((_NBUF, _CH, 128), jnp.float32),
    ] + [pltpu.SemaphoreType.DMA] * (2 * _NBUF),
)


@jax.jit
def kernel(x, embedding):
    idx = x.astype(jnp.int32).reshape(_NW, _NCHUNK, _CH)
    table = _tc_pad(jnp.transpose(embedding))
    out = _gather(table, idx)
    return out.reshape(_B, _H, 128)[:, :, :_D]
